# Initial kernel scaffold; baseline (speedup 1.0000x reference)
import jax
import jax.numpy as jnp
from jax.experimental import pallas as pl


def kernel(x0, x1, x2, x3, boxes):
    raise NotImplementedError("write your pallas kernel here")


# speedup vs baseline: 33.5661x; 33.5661x over previous
import functools

import jax
import jax.numpy as jnp
from jax import lax
from jax.experimental import pallas as pl
from jax.experimental.pallas import tpu as pltpu
from jax.experimental.pallas import tpu_sc as plsc

OUT = 7
C = 256
WS = (256, 128, 64, 32)
LVL_SCALES = (0.25, 0.125, 0.0625, 0.03125)
NBATCH = 2
M = 512
NW = 32
RPW = M // NW
BAGS = RPW * OUT
BAG_ROWS = OUT * 16

_BASES = []
_acc = 0
for _w in WS:
    _BASES.append(_acc)
    _acc += NBATCH * _w * _w
TABLE_ROWS = _acc

T3 = (224.0 * (0.5 - 1e-8)) ** 2
T4 = (224.0 * (1.0 - 1e-8)) ** 2
T5 = (224.0 * (2.0 - 1e-8)) ** 2

_i32 = jnp.int32
_f32 = jnp.float32


def _splat_i(v):
    return jnp.full((16,), v, _i32)


def _body(table, boxesr, out, boxes_v, sampi, sampf, idx_scr, w_scr, rowbuf,
          outbuf, gs0, gs1, os0, os1):
    gsems = (gs0, gs1)
    osems = (os0, os1)
    wid = lax.axis_index("s") * 2 + lax.axis_index("c")
    m_base = wid * RPW
    pltpu.sync_copy(boxesr, boxes_v)

    lanes = lax.iota(_i32, 16)
    lanef = lanes.astype(_f32)
    bit3 = (lanes >> 3) & 1
    bit2 = (lanes >> 2) & 1
    nymask = ((lanes >> 1) & 1) == 1
    nxmask = (lanes & 1) == 1
    offv = (lanef + 0.5) * 0.5
    lane_lt14 = lanes < 14

    def axis_samples(lo, hi, scale, wf, wi):
        lof = lo * scale - 0.5
        hif = hi * scale - 0.5
        bsz = (hif - lof) / 7.0
        s = lof + offv * bsz
        valid = (s > -1.0) & (s < wf) & lane_lt14
        sc = jnp.clip(s, 0.0, wf - 1.0)
        i0 = sc.astype(_i32)
        frac = sc - i0.astype(_f32)
        vf = jnp.where(valid, 1.0, 0.0).astype(_f32)
        w_hi = (1.0 - frac) * vf
        w_lo = frac * vf
        i1 = jnp.minimum(i0 + 1, wi - 1)
        return i0, i1, w_hi, w_lo

    def build_and_issue(t, slot):
        m = m_base + t // OUT
        i = lax.rem(t, OUT)
        msp = jnp.full((16,), m, _i32)
        bx1 = plsc.load_gather(boxes_v, [_splat_i(0), msp])
        by1 = plsc.load_gather(boxes_v, [_splat_i(1), msp])
        bx2 = plsc.load_gather(boxes_v, [_splat_i(2), msp])
        by2 = plsc.load_gather(boxes_v, [_splat_i(3), msp])

        area = (bx2 - bx1) * (by2 - by1)
        ge3 = area >= T3
        ge4 = area >= T4
        ge5 = area >= T5
        scale = jnp.where(ge5, LVL_SCALES[3],
                          jnp.where(ge4, LVL_SCALES[2],
                                    jnp.where(ge3, LVL_SCALES[1],
                                              LVL_SCALES[0])))
        wi = jnp.where(ge5, WS[3],
                       jnp.where(ge4, WS[2], jnp.where(ge3, WS[1], WS[0])))
        base = jnp.where(ge5, _BASES[3],
                         jnp.where(ge4, _BASES[2],
                                   jnp.where(ge3, _BASES[1], _BASES[0])))
        rb = base + jnp.where(m >= 256, wi * wi, 0)
        wf = wi.astype(_f32)

        y0i, y1i, hy, ly = axis_samples(by1, by2, scale, wf, wi)
        x0i, x1i, hx, lx = axis_samples(bx1, bx2, scale, wf, wi)
        sampi[0, :] = y0i
        sampi[1, :] = y1i
        sampi[2, :] = x0i
        sampi[3, :] = x1i
        sampf[0, :] = hy
        sampf[1, :] = ly
        sampf[2, :] = hx
        sampf[3, :] = lx

        ysel = 2 * jnp.full((16,), i, _i32) + bit3
        y0g = plsc.load_gather(sampi, [_splat_i(0), ysel])
        y1g = plsc.load_gather(sampi, [_splat_i(1), ysel])
        hyg = plsc.load_gather(sampf, [_splat_i(0), ysel])
        lyg = plsc.load_gather(sampf, [_splat_i(1), ysel])
        yy = jnp.where(nymask, y1g, y0g)
        wy = jnp.where(nymask, lyg, hyg)
        ybase = rb + yy * wi
        for j in range(OUT):
            xsel = 2 * j + bit2
            x0g = plsc.load_gather(sampi, [_splat_i(2), xsel])
            x1g = plsc.load_gather(sampi, [_splat_i(3), xsel])
            hxg = plsc.load_gather(sampf, [_splat_i(2), xsel])
            lxg = plsc.load_gather(sampf, [_splat_i(3), xsel])
            xx = jnp.where(nxmask, x1g, x0g)
            wx = jnp.where(nxmask, lxg, hxg)
            idx_scr[slot, j * 16:(j + 1) * 16] = ybase + xx
            w_scr[slot, j * 16:(j + 1) * 16] = wy * wx * 0.25
        pltpu.make_async_copy(table.at[idx_scr.at[slot]], rowbuf.at[slot],
                              gsems[slot]).start()

    def compute(slot):
        def binbody(j, carry):
            wks = [plsc.load_gather(w_scr.at[slot], [jnp.full((16,), j * 16 + k, _i32)])
                   for k in range(16)]
            for c in range(16):
                acc = rowbuf[slot, j * 16, c * 16:(c + 1) * 16] * wks[0]
                for k in range(1, 16):
                    acc = acc + rowbuf[slot, j * 16 + k, c * 16:(c + 1) * 16] * wks[k]
                outbuf[slot, j, c * 16:(c + 1) * 16] = acc
            return carry
        lax.fori_loop(0, OUT, binbody, 0)

    def issue_out(t, slot):
        m = m_base + t // OUT
        i = lax.rem(t, OUT)
        row0 = m * (OUT * OUT) + i * OUT
        pltpu.make_async_copy(outbuf.at[slot], out.at[pl.ds(row0, OUT)],
                              osems[slot]).start()

    build_and_issue(0, 0)
    build_and_issue(1, 1)

    def kbody(kk, carry):
        for b in range(2):
            t = 2 * kk + b
            pltpu.make_async_copy(table.at[idx_scr.at[b]], rowbuf.at[b],
                                  gsems[b]).wait()

            @pl.when(kk >= 1)
            def _wait_prev_out():
                pltpu.make_async_copy(outbuf.at[b], out.at[pl.ds(0, OUT)],
                                      osems[b]).wait()

            compute(b)
            issue_out(t, b)

            @pl.when(t + 2 < BAGS)
            def _issue_next():
                build_and_issue(t + 2, b)
        return carry

    lax.fori_loop(0, BAGS // 2, kbody, 0)
    pltpu.make_async_copy(outbuf.at[0], out.at[pl.ds(0, OUT)], os0).wait()
    pltpu.make_async_copy(outbuf.at[1], out.at[pl.ds(0, OUT)], os1).wait()


_mesh = plsc.VectorSubcoreMesh(core_axis_name="c", subcore_axis_name="s")

_sc_call = functools.partial(
    pl.kernel,
    mesh=_mesh,
    compiler_params=pltpu.CompilerParams(use_tc_tiling_on_sc=False,
                                         needs_layout_passes=False),
    out_type=jax.ShapeDtypeStruct((M * OUT * OUT, C), _f32),
    scratch_types=[
        pltpu.VMEM((4, M), _f32),
        pltpu.VMEM((4, 16), _i32),
        pltpu.VMEM((4, 16), _f32),
        pltpu.VMEM((2, BAG_ROWS), _i32),
        pltpu.VMEM((2, BAG_ROWS), _f32),
        pltpu.VMEM((2, BAG_ROWS, C), _f32),
        pltpu.VMEM((2, OUT, C), _f32),
        pltpu.SemaphoreType.DMA,
        pltpu.SemaphoreType.DMA,
        pltpu.SemaphoreType.DMA,
        pltpu.SemaphoreType.DMA,
    ],
)(_body)


def kernel(x0, x1, x2, x3, boxes):
    tabs = [f.transpose(0, 2, 3, 1).reshape(-1, C) for f in (x0, x1, x2, x3)]
    table = jnp.concatenate(tabs, axis=0)
    boxesr = boxes.reshape(M, 4).T
    out = _sc_call(table, boxesr)
    return out.reshape(M, OUT, OUT, C).transpose(0, 3, 1, 2)


# speedup vs baseline: 39.5494x; 1.1783x over previous
import functools

import jax
import jax.numpy as jnp
from jax import lax
from jax.experimental import pallas as pl
from jax.experimental.pallas import tpu as pltpu
from jax.experimental.pallas import tpu_sc as plsc

OUT = 7
C = 256
WS = (256, 128, 64, 32)
LVL_SCALES = (0.25, 0.125, 0.0625, 0.03125)
NBATCH = 2
M = 512
NW = 32
RPW = M // NW
BAGS = RPW * OUT
BAG_ROWS = OUT * 16
NBUF = 4

T3 = (224.0 * (0.5 - 1e-8)) ** 2
T4 = (224.0 * (1.0 - 1e-8)) ** 2
T5 = (224.0 * (2.0 - 1e-8)) ** 2

_i32 = jnp.int32
_f32 = jnp.float32


def _splat_i(v):
    return jnp.full((16,), v, _i32)


def _body(t0, t1, t2, t3, boxesr, out, boxes_v, sampi, sampf, idx_scr, w_scr,
          rowbuf, outbuf, *sems):
    tables = (t0, t1, t2, t3)
    gsems = sems[:NBUF]
    osems = sems[NBUF:]
    wid = lax.axis_index("s") * 2 + lax.axis_index("c")
    m_base = wid * RPW
    pltpu.sync_copy(boxesr, boxes_v)

    lanes = lax.iota(_i32, 16)
    lanef = lanes.astype(_f32)
    bit3 = (lanes >> 3) & 1
    bit2 = (lanes >> 2) & 1
    nymask = ((lanes >> 1) & 1) == 1
    nxmask = (lanes & 1) == 1
    offv = (lanef + 0.5) * 0.5
    lane_lt14 = lanes < 14

    def axis_samples(lo, hi, scale, wf, wi):
        lof = lo * scale - 0.5
        hif = hi * scale - 0.5
        bsz = (hif - lof) / 7.0
        s = lof + offv * bsz
        valid = (s > -1.0) & (s < wf) & lane_lt14
        sc = jnp.clip(s, 0.0, wf - 1.0)
        i0 = sc.astype(_i32)
        frac = sc - i0.astype(_f32)
        vf = jnp.where(valid, 1.0, 0.0).astype(_f32)
        w_hi = (1.0 - frac) * vf
        w_lo = frac * vf
        i1 = jnp.minimum(i0 + 1, wi - 1)
        return i0, i1, w_hi, w_lo

    def build_and_issue(t, slot):
        m = m_base + t // OUT
        i = lax.rem(t, OUT)
        msp = jnp.full((16,), m, _i32)
        bx1 = plsc.load_gather(boxes_v, [_splat_i(0), msp])
        by1 = plsc.load_gather(boxes_v, [_splat_i(1), msp])
        bx2 = plsc.load_gather(boxes_v, [_splat_i(2), msp])
        by2 = plsc.load_gather(boxes_v, [_splat_i(3), msp])

        area = (bx2 - bx1) * (by2 - by1)
        ge3 = area >= T3
        ge4 = area >= T4
        ge5 = area >= T5
        scale = jnp.where(ge5, LVL_SCALES[3],
                          jnp.where(ge4, LVL_SCALES[2],
                                    jnp.where(ge3, LVL_SCALES[1],
                                              LVL_SCALES[0])))
        wi = jnp.where(ge5, WS[3],
                       jnp.where(ge4, WS[2], jnp.where(ge3, WS[1], WS[0])))
        lvl = (ge3.astype(_i32) + ge4.astype(_i32) + ge5.astype(_i32))
        rb = jnp.where(m >= 256, wi * wi, 0)
        wf = wi.astype(_f32)

        y0i, y1i, hy, ly = axis_samples(by1, by2, scale, wf, wi)
        x0i, x1i, hx, lx = axis_samples(bx1, bx2, scale, wf, wi)
        sampi[0, :] = y0i
        sampi[1, :] = y1i
        sampi[2, :] = x0i
        sampi[3, :] = x1i
        sampf[0, :] = hy
        sampf[1, :] = ly
        sampf[2, :] = hx
        sampf[3, :] = lx

        ysel = 2 * jnp.full((16,), i, _i32) + bit3
        y0g = plsc.load_gather(sampi, [_splat_i(0), ysel])
        y1g = plsc.load_gather(sampi, [_splat_i(1), ysel])
        hyg = plsc.load_gather(sampf, [_splat_i(0), ysel])
        lyg = plsc.load_gather(sampf, [_splat_i(1), ysel])
        yy = jnp.where(nymask, y1g, y0g)
        wy = jnp.where(nymask, lyg, hyg)
        ybase = rb + yy * wi
        for j in range(OUT):
            xsel = 2 * j + bit2
            x0g = plsc.load_gather(sampi, [_splat_i(2), xsel])
            x1g = plsc.load_gather(sampi, [_splat_i(3), xsel])
            hxg = plsc.load_gather(sampf, [_splat_i(2), xsel])
            lxg = plsc.load_gather(sampf, [_splat_i(3), xsel])
            xx = jnp.where(nxmask, x1g, x0g)
            wx = jnp.where(nxmask, lxg, hxg)
            idx_scr[slot, j * 16:(j + 1) * 16] = ybase + xx
            w_scr[slot, j * 16:(j + 1) * 16] = wy * wx * 0.25
        lvl_s = jnp.max(lvl)
        branches = [
            (lambda tab: lambda: pltpu.make_async_copy(
                tab.at[idx_scr.at[slot]], rowbuf.at[slot],
                gsems[slot]).start())(tab)
            for tab in tables
        ]
        lax.switch(lvl_s, branches)

    def compute(slot):
        def binbody(j, carry):
            wks = [plsc.load_gather(w_scr.at[slot],
                                    [jnp.full((16,), j * 16 + k, _i32)])
                   for k in range(16)]
            for c in range(16):
                acc = rowbuf[slot, j * 16, c * 16:(c + 1) * 16] * wks[0]
                for k in range(1, 16):
                    acc = acc + rowbuf[slot, j * 16 + k,
                                       c * 16:(c + 1) * 16] * wks[k]
                outbuf[slot, j, c * 16:(c + 1) * 16] = acc
            return carry
        lax.fori_loop(0, OUT, binbody, 0)

    def issue_out(t, slot):
        m = m_base + t // OUT
        i = lax.rem(t, OUT)
        row0 = m * (OUT * OUT) + i * OUT
        pltpu.make_async_copy(outbuf.at[slot], out.at[pl.ds(row0, OUT)],
                              osems[slot]).start()

    for s in range(NBUF):
        build_and_issue(s, s)

    def kbody(kk, carry):
        for b in range(NBUF):
            t = NBUF * kk + b
            pltpu.make_async_copy(t0.at[idx_scr.at[b]], rowbuf.at[b],
                                  gsems[b]).wait()

            @pl.when(kk >= 1)
            def _wait_prev_out():
                pltpu.make_async_copy(outbuf.at[b], out.at[pl.ds(0, OUT)],
                                      osems[b]).wait()

            compute(b)
            issue_out(t, b)

            @pl.when(t + NBUF < BAGS)
            def _issue_next():
                build_and_issue(t + NBUF, b)
        return carry

    lax.fori_loop(0, BAGS // NBUF, kbody, 0)
    for s in range(NBUF):
        pltpu.make_async_copy(outbuf.at[s], out.at[pl.ds(0, OUT)],
                              osems[s]).wait()


_mesh = plsc.VectorSubcoreMesh(core_axis_name="c", subcore_axis_name="s")

_sc_call = functools.partial(
    pl.kernel,
    mesh=_mesh,
    compiler_params=pltpu.CompilerParams(use_tc_tiling_on_sc=False,
                                         needs_layout_passes=False),
    out_type=jax.ShapeDtypeStruct((M * OUT * OUT, C), _f32),
    scratch_types=[
        pltpu.VMEM((4, M), _f32),
        pltpu.VMEM((4, 16), _i32),
        pltpu.VMEM((4, 16), _f32),
        pltpu.VMEM((NBUF, BAG_ROWS), _i32),
        pltpu.VMEM((NBUF, BAG_ROWS), _f32),
        pltpu.VMEM((NBUF, BAG_ROWS, C), _f32),
        pltpu.VMEM((NBUF, OUT, C), _f32),
    ] + [pltpu.SemaphoreType.DMA] * (2 * NBUF),
)(_body)


def kernel(x0, x1, x2, x3, boxes):
    tabs = [f.transpose(0, 2, 3, 1).reshape(-1, C) for f in (x0, x1, x2, x3)]
    boxesr = boxes.reshape(M, 4).T
    out = _sc_call(*tabs, boxesr)
    return out.reshape(M, OUT, OUT, C).transpose(0, 3, 1, 2)


# speedup vs baseline: 45.2916x; 1.1452x over previous
import functools

import jax
import jax.numpy as jnp
from jax import lax
from jax.experimental import pallas as pl
from jax.experimental.pallas import tpu as pltpu
from jax.experimental.pallas import tpu_sc as plsc

OUT = 7
C = 256
WS = (256, 128, 64, 32)
LVL_SCALES = (0.25, 0.125, 0.0625, 0.03125)
NBATCH = 2
M = 512
NW = 32
RPW = M // NW
BAGS = RPW * OUT
BAG_ROWS = OUT * 16
NBUF = 4

T3 = (224.0 * (0.5 - 1e-8)) ** 2
T4 = (224.0 * (1.0 - 1e-8)) ** 2
T5 = (224.0 * (2.0 - 1e-8)) ** 2

_i32 = jnp.int32
_f32 = jnp.float32


def _splat_i(v):
    return jnp.full((16,), v, _i32)


def _body(t0, t1, t2, t3, boxesr, out, boxes_v, sampi, sampf, idx_scr, w_scr,
          rowbuf, outbuf, *sems):
    tables = (t0, t1, t2, t3)
    gsems = sems[:NBUF]
    osems = sems[NBUF:]
    wid = lax.axis_index("s") * 2 + lax.axis_index("c")
    m_base = wid * RPW
    pltpu.sync_copy(boxesr, boxes_v)

    lanes = lax.iota(_i32, 16)
    lanef = lanes.astype(_f32)
    bit3 = (lanes >> 3) & 1
    bit2 = (lanes >> 2) & 1
    nymask = ((lanes >> 1) & 1) == 1
    nxmask = (lanes & 1) == 1
    offv = (lanef + 0.5) * 0.5
    lane_lt14 = lanes < 14

    def axis_samples(lo, hi, scale, wf, wi):
        lof = lo * scale - 0.5
        hif = hi * scale - 0.5
        bsz = (hif - lof) / 7.0
        s = lof + offv * bsz
        valid = (s > -1.0) & (s < wf) & lane_lt14
        sc = jnp.clip(s, 0.0, wf - 1.0)
        i0 = sc.astype(_i32)
        frac = sc - i0.astype(_f32)
        vf = jnp.where(valid, 1.0, 0.0).astype(_f32)
        w_hi = (1.0 - frac) * vf
        w_lo = frac * vf
        i1 = jnp.minimum(i0 + 1, wi - 1)
        return i0, i1, w_hi, w_lo

    def build_and_issue(t, slot):
        m = m_base + t // OUT
        i = lax.rem(t, OUT)
        msp = jnp.full((16,), m, _i32)
        bx1 = plsc.load_gather(boxes_v, [_splat_i(0), msp])
        by1 = plsc.load_gather(boxes_v, [_splat_i(1), msp])
        bx2 = plsc.load_gather(boxes_v, [_splat_i(2), msp])
        by2 = plsc.load_gather(boxes_v, [_splat_i(3), msp])

        area = (bx2 - bx1) * (by2 - by1)
        ge3 = area >= T3
        ge4 = area >= T4
        ge5 = area >= T5
        scale = jnp.where(ge5, LVL_SCALES[3],
                          jnp.where(ge4, LVL_SCALES[2],
                                    jnp.where(ge3, LVL_SCALES[1],
                                              LVL_SCALES[0])))
        wi = jnp.where(ge5, WS[3],
                       jnp.where(ge4, WS[2], jnp.where(ge3, WS[1], WS[0])))
        lvl = (ge3.astype(_i32) + ge4.astype(_i32) + ge5.astype(_i32))
        rb = jnp.where(m >= 256, wi * wi, 0)
        wf = wi.astype(_f32)

        y0i, y1i, hy, ly = axis_samples(by1, by2, scale, wf, wi)
        x0i, x1i, hx, lx = axis_samples(bx1, bx2, scale, wf, wi)
        sampi[0, :] = y0i
        sampi[1, :] = y1i
        sampi[2, :] = x0i
        sampi[3, :] = x1i
        sampf[0, :] = hy
        sampf[1, :] = ly
        sampf[2, :] = hx
        sampf[3, :] = lx

        ysel = 2 * jnp.full((16,), i, _i32) + bit3
        y0g = plsc.load_gather(sampi, [_splat_i(0), ysel])
        y1g = plsc.load_gather(sampi, [_splat_i(1), ysel])
        hyg = plsc.load_gather(sampf, [_splat_i(0), ysel])
        lyg = plsc.load_gather(sampf, [_splat_i(1), ysel])
        yy = jnp.where(nymask, y1g, y0g)
        wy = jnp.where(nymask, lyg, hyg)
        ybase = rb + yy * wi
        for j in range(OUT):
            xsel = 2 * j + bit2
            x0g = plsc.load_gather(sampi, [_splat_i(2), xsel])
            x1g = plsc.load_gather(sampi, [_splat_i(3), xsel])
            hxg = plsc.load_gather(sampf, [_splat_i(2), xsel])
            lxg = plsc.load_gather(sampf, [_splat_i(3), xsel])
            xx = jnp.where(nxmask, x1g, x0g)
            wx = jnp.where(nxmask, lxg, hxg)
            idx_scr[slot, j * 16:(j + 1) * 16] = ybase + xx
            w_scr[slot, j * 16:(j + 1) * 16] = wy * wx * 0.25
        lvl_s = jnp.max(lvl)
        branches = [
            (lambda tab: lambda: pltpu.make_async_copy(
                tab.at[idx_scr.at[slot]], rowbuf.at[slot],
                gsems[slot]).start())(tab)
            for tab in tables
        ]
        lax.switch(lvl_s, branches)

    def compute(slot):
        def binbody(j, carry):
            wks = [plsc.load_gather(w_scr.at[slot],
                                    [jnp.full((16,), j * 16 + k, _i32)])
                   for k in range(16)]
            for c in range(8):
                lo, hi = plsc.unpack(
                    rowbuf[slot, j * 16, c * 32:(c + 1) * 32],
                    format=plsc.PackFormat.INTERLEAVED)
                acc_lo = lo * wks[0]
                acc_hi = hi * wks[0]
                for k in range(1, 16):
                    lo, hi = plsc.unpack(
                        rowbuf[slot, j * 16 + k, c * 32:(c + 1) * 32],
                        format=plsc.PackFormat.INTERLEAVED)
                    acc_lo = acc_lo + lo * wks[k]
                    acc_hi = acc_hi + hi * wks[k]
                outbuf[slot, j, c * 32:c * 32 + 16] = acc_lo
                outbuf[slot, j, c * 32 + 16:(c + 1) * 32] = acc_hi
            return carry
        lax.fori_loop(0, OUT, binbody, 0)

    def issue_out(t, slot):
        m = m_base + t // OUT
        i = lax.rem(t, OUT)
        row0 = m * (OUT * OUT) + i * OUT
        pltpu.make_async_copy(outbuf.at[slot], out.at[pl.ds(row0, OUT)],
                              osems[slot]).start()

    for s in range(NBUF):
        build_and_issue(s, s)

    def kbody(kk, carry):
        for b in range(NBUF):
            t = NBUF * kk + b
            pltpu.make_async_copy(t0.at[idx_scr.at[b]], rowbuf.at[b],
                                  gsems[b]).wait()

            @pl.when(kk >= 1)
            def _wait_prev_out():
                pltpu.make_async_copy(outbuf.at[b], out.at[pl.ds(0, OUT)],
                                      osems[b]).wait()

            compute(b)
            issue_out(t, b)

            @pl.when(t + NBUF < BAGS)
            def _issue_next():
                build_and_issue(t + NBUF, b)
        return carry

    lax.fori_loop(0, BAGS // NBUF, kbody, 0)
    for s in range(NBUF):
        pltpu.make_async_copy(outbuf.at[s], out.at[pl.ds(0, OUT)],
                              osems[s]).wait()


_mesh = plsc.VectorSubcoreMesh(core_axis_name="c", subcore_axis_name="s")

_sc_call = functools.partial(
    pl.kernel,
    mesh=_mesh,
    compiler_params=pltpu.CompilerParams(use_tc_tiling_on_sc=False,
                                         needs_layout_passes=False),
    out_type=jax.ShapeDtypeStruct((M * OUT * OUT, C), _f32),
    scratch_types=[
        pltpu.VMEM((4, M), _f32),
        pltpu.VMEM((4, 16), _i32),
        pltpu.VMEM((4, 16), _f32),
        pltpu.VMEM((NBUF, BAG_ROWS), _i32),
        pltpu.VMEM((NBUF, BAG_ROWS), _f32),
        pltpu.VMEM((NBUF, BAG_ROWS, C), jnp.bfloat16),
        pltpu.VMEM((NBUF, OUT, C), _f32),
    ] + [pltpu.SemaphoreType.DMA] * (2 * NBUF),
)(_body)


_PERM = []
for _b in range(0, C, 32):
    for _i in range(16):
        _PERM.extend((_b + _i, _b + 16 + _i))
_PERM = tuple(_PERM)


def kernel(x0, x1, x2, x3, boxes):
    perm = jnp.asarray(_PERM, jnp.int32)
    tabs = [f[:, perm].transpose(0, 2, 3, 1).reshape(-1, C).astype(jnp.bfloat16)
            for f in (x0, x1, x2, x3)]
    boxesr = boxes.reshape(M, 4).T
    out = _sc_call(*tabs, boxesr)
    return out.reshape(M, OUT, OUT, C).transpose(0, 3, 1, 2)
